# baseline (speedup 1.0000x reference)
import functools

import jax
import jax.numpy as jnp
from jax import lax
from jax.experimental import pallas as pl
from jax.experimental.pallas import tpu as pltpu
from jax.experimental.pallas import tpu_sc as plsc

N = 10000
NP = 10240
E = 320000
D = 128
NC = 2
NS = 16
NW = NC * NS
RPW = D // NW
PRPW = RPW // 2
EPW = E // NW
CHUNK = 3200
NCHUNK = E // CHUNK
GROUPS = CHUNK // 16
BLK = 1024


def _unpack_edges(ev):
    e16 = plsc.bitcast(ev, jnp.int16)
    sv, dv = plsc.unpack(e16, format=plsc.PackFormat.INTERLEAVED,
                         preferred_element_type=jnp.int32)
    return sv, dv


def _sc_agg_body(with_counts, *refs):
    if with_counts:
        (yt, edges, aggt, cpart,
         yv, accv, ev, cntv, cev,
         sem0, sem1) = refs
    else:
        (yt, edges, aggt,
         yv, accv, ev,
         sem0, sem1) = refs
    sems = (sem0, sem1)
    wid = lax.axis_index("s") * NC + lax.axis_index("c")

    for b in (0, 1):
        pltpu.async_copy(edges.at[pl.ds(b * CHUNK, CHUNK)], ev.at[b], sems[b])

    for r in range(PRPW):
        pltpu.sync_copy(yt.at[PRPW * wid + r], yv.at[pl.ds(r * NP, NP)])

    zf = jnp.zeros((16,), jnp.float32)

    @plsc.parallel_loop(0, (RPW * NP) // 16, unroll=8)
    def zbody(i):
        accv[pl.ds(i * 16, 16)] = zf

    def process(b):
        @plsc.parallel_loop(0, GROUPS, unroll=8)
        def gbody(g):
            sv, dv = _unpack_edges(ev[b, pl.ds(g * 16, 16)])
            for r in range(PRPW):
                pv = plsc.load_gather(yv, [sv + (r * NP)])
                lo, hi = plsc.unpack(plsc.bitcast(pv, jnp.bfloat16),
                                     format=plsc.PackFormat.INTERLEAVED)
                plsc.addupdate_scatter(accv, [dv + (2 * r * NP)], lo)
                plsc.addupdate_scatter(accv, [dv + ((2 * r + 1) * NP)], hi)

    def pair(p, _):
        for b in (0, 1):
            c = 2 * p + b
            pltpu.make_async_copy(edges.at[pl.ds(0, CHUNK)], ev.at[b], sems[b]).wait()
            process(b)

            @pl.when(c + 2 < NCHUNK)
            def _():
                off = (c + 2) * CHUNK
                pltpu.async_copy(edges.at[pl.ds(off, CHUNK)], ev.at[b], sems[b])

        return 0

    lax.fori_loop(0, NCHUNK // 2, pair, 0)

    for r in range(RPW):
        pltpu.sync_copy(accv.at[pl.ds(r * NP, NP)], aggt.at[RPW * wid + r])

    if with_counts:
        pltpu.sync_copy(edges.at[pl.ds(EPW * wid, EPW)], cev)

        @plsc.parallel_loop(0, NP // 16, unroll=8)
        def czero(i):
            cntv[pl.ds(i * 16, 16)] = zf

        ones = jnp.full((16,), 1.0, jnp.float32)

        @plsc.parallel_loop(0, EPW // 16, unroll=4)
        def cbody(g):
            _, dv = _unpack_edges(cev[pl.ds(g * 16, 16)])
            plsc.addupdate_scatter(cntv, [dv], ones)

        pltpu.sync_copy(cntv, cpart.at[wid])


def _make_sc_agg(with_counts):
    mesh = plsc.VectorSubcoreMesh(core_axis_name="c", subcore_axis_name="s")
    if with_counts:
        out_type = (jax.ShapeDtypeStruct((D, NP), jnp.float32),
                    jax.ShapeDtypeStruct((NW, NP), jnp.float32))
    else:
        out_type = jax.ShapeDtypeStruct((D, NP), jnp.float32)
    scratch = [
        pltpu.VMEM((PRPW * NP,), jnp.int32),
        pltpu.VMEM((RPW * NP,), jnp.float32),
        pltpu.VMEM((2, CHUNK), jnp.int32),
    ]
    if with_counts:
        scratch.append(pltpu.VMEM((NP,), jnp.float32))
        scratch.append(pltpu.VMEM((EPW,), jnp.int32))
    scratch += [pltpu.SemaphoreType.DMA] * 2
    return pl.kernel(
        functools.partial(_sc_agg_body, with_counts),
        mesh=mesh,
        out_type=out_type,
        scratch_types=scratch,
        compiler_params=pltpu.CompilerParams(needs_layout_passes=False),
    )


_sc_agg_counts = _make_sc_agg(True)
_sc_agg = _make_sc_agg(False)


def _pack_block(y):
    u = lax.bitcast_convert_type(y.astype(jnp.bfloat16),
                                 jnp.uint16).astype(jnp.uint32)
    ur = u.reshape(D // 2, 2, u.shape[-1])
    w = ur[:, 0, :] | (ur[:, 1, :] << 16)
    return lax.bitcast_convert_type(w, jnp.int32)


def _pre_body(wl_ref, x_ref, o_ref):
    dn = (((1,), (1,)), ((), ()))
    y = lax.dot_general(wl_ref[...], x_ref[...], dn,
                        preferred_element_type=jnp.float32)
    o_ref[...] = _pack_block(y)


def _pre(W_l, x):
    return pl.pallas_call(
        _pre_body,
        grid=(NP // BLK,),
        in_specs=[
            pl.BlockSpec((D, D), lambda i: (0, 0)),
            pl.BlockSpec((BLK, D), lambda i: (i, 0)),
        ],
        out_specs=pl.BlockSpec((D // 2, BLK), lambda i: (0, i)),
        out_shape=jax.ShapeDtypeStruct((D // 2, NP), jnp.int32),
    )(W_l, x)


def _eye():
    r = lax.broadcasted_iota(jnp.int32, (D, D), 0)
    c = lax.broadcasted_iota(jnp.int32, (D, D), 1)
    return (r == c).astype(jnp.float32)


def _mid_body(agg_ref, cp_ref, x_ref, wr_ref, b_ref, w2l_ref, h_ref, y2_ref):
    cnt = jnp.sum(cp_ref[...], axis=0, keepdims=True)
    inv = 1.0 / jnp.maximum(cnt, 1.0)
    t = agg_ref[...] * inv
    dn0 = (((0,), (0,)), ((), ()))
    dn1 = (((1,), (1,)), ((), ()))
    mean = lax.dot_general(t, _eye(), dn0, preferred_element_type=jnp.float32)
    h = mean + lax.dot_general(x_ref[...], wr_ref[...], dn1,
                               preferred_element_type=jnp.float32)
    h = jnp.maximum(h + b_ref[...], 0.0)
    h_ref[...] = h
    y2 = lax.dot_general(w2l_ref[...], h, dn1,
                         preferred_element_type=jnp.float32)
    y2_ref[...] = _pack_block(y2)


def _mid(aggT, cpart, x, W_r, b, W2_l):
    return pl.pallas_call(
        _mid_body,
        grid=(NP // BLK,),
        in_specs=[
            pl.BlockSpec((D, BLK), lambda i: (0, i)),
            pl.BlockSpec((NW, BLK), lambda i: (0, i)),
            pl.BlockSpec((BLK, D), lambda i: (i, 0)),
            pl.BlockSpec((D, D), lambda i: (0, 0)),
            pl.BlockSpec((1, D), lambda i: (0, 0)),
            pl.BlockSpec((D, D), lambda i: (0, 0)),
        ],
        out_specs=[
            pl.BlockSpec((BLK, D), lambda i: (i, 0)),
            pl.BlockSpec((D // 2, BLK), lambda i: (0, i)),
        ],
        out_shape=[
            jax.ShapeDtypeStruct((N, D), jnp.float32),
            jax.ShapeDtypeStruct((D // 2, NP), jnp.int32),
        ],
    )(aggT, cpart, x, W_r, b.reshape(1, D), W2_l)


def _post_body(agg_ref, cp_ref, x_ref, wr_ref, b_ref, o_ref):
    cnt = jnp.sum(cp_ref[...], axis=0, keepdims=True)
    inv = 1.0 / jnp.maximum(cnt, 1.0)
    t = agg_ref[...] * inv
    dn0 = (((0,), (0,)), ((), ()))
    dn1 = (((1,), (1,)), ((), ()))
    mean = lax.dot_general(t, _eye(), dn0, preferred_element_type=jnp.float32)
    h = mean + lax.dot_general(x_ref[...], wr_ref[...], dn1,
                               preferred_element_type=jnp.float32)
    o_ref[...] = jnp.maximum(h + b_ref[...], 0.0)


def _post(aggT, cpart, h1, W_r, b):
    return pl.pallas_call(
        _post_body,
        grid=(NP // BLK,),
        in_specs=[
            pl.BlockSpec((D, BLK), lambda i: (0, i)),
            pl.BlockSpec((NW, BLK), lambda i: (0, i)),
            pl.BlockSpec((BLK, D), lambda i: (i, 0)),
            pl.BlockSpec((D, D), lambda i: (0, 0)),
            pl.BlockSpec((1, D), lambda i: (0, 0)),
        ],
        out_specs=pl.BlockSpec((BLK, D), lambda i: (i, 0)),
        out_shape=jax.ShapeDtypeStruct((N, D), jnp.float32),
    )(aggT, cpart, h1, W_r, b.reshape(1, D))


def kernel(x, edge_index, W1_l, b1_l, W1_r, W2_l, b2_l, W2_r):
    src = edge_index[0].astype(jnp.int32)
    dst = edge_index[1].astype(jnp.int32)
    edges = src | (dst << 16)
    y1p = _pre(W1_l, x)
    agg1T, cpart = _sc_agg_counts(y1p, edges)
    h1, y2p = _mid(agg1T, cpart, x, W1_r, b1_l, W2_l)
    agg2T = _sc_agg(y2p, edges)
    return _post(agg2T, cpart, h1, W2_r, b2_l)


# speedup vs baseline: 1.0070x; 1.0070x over previous
import functools

import jax
import jax.numpy as jnp
from jax import lax
from jax.experimental import pallas as pl
from jax.experimental.pallas import tpu as pltpu
from jax.experimental.pallas import tpu_sc as plsc

N = 10000
NP = 10240
E = 320000
D = 128
NC = 2
NS = 16
NW = NC * NS
RPW = D // NW
PRPW = RPW // 2
EPW = E // NW
CHUNK = 3200
NCHUNK = E // CHUNK
GROUPS = CHUNK // 16
BLK = 1024


def _unpack_edges(ev):
    e16 = plsc.bitcast(ev, jnp.int16)
    sv, dv = plsc.unpack(e16, format=plsc.PackFormat.INTERLEAVED,
                         preferred_element_type=jnp.int32)
    return sv, dv


def _sc_agg_body(with_counts, *refs):
    if with_counts:
        (yt, edges, aggt, cpart,
         yv, accv, ev, cntv, cev,
         sem0, sem1) = refs
    else:
        (yt, edges, aggt,
         yv, accv, ev,
         sem0, sem1) = refs
    sems = (sem0, sem1)
    wid = lax.axis_index("s") * NC + lax.axis_index("c")

    for b in (0, 1):
        pltpu.async_copy(edges.at[pl.ds(b * CHUNK, CHUNK)], ev.at[b], sems[b])

    for r in range(PRPW):
        pltpu.sync_copy(yt.at[PRPW * wid + r], yv.at[pl.ds(r * NP, NP)])

    zf = jnp.zeros((16,), jnp.float32)

    @plsc.parallel_loop(0, (RPW * NP) // 16, unroll=8)
    def zbody(i):
        accv[pl.ds(i * 16, 16)] = zf

    def process(b):
        @plsc.parallel_loop(0, GROUPS, unroll=4)
        def gbody(g):
            sv, dv = _unpack_edges(ev[b, pl.ds(g * 16, 16)])
            for r in range(PRPW):
                pv = plsc.load_gather(yv, [sv + (r * NP)])
                lo, hi = plsc.unpack(plsc.bitcast(pv, jnp.bfloat16),
                                     format=plsc.PackFormat.INTERLEAVED)
                plsc.addupdate_scatter(accv, [dv + (2 * r * NP)], lo)
                plsc.addupdate_scatter(accv, [dv + ((2 * r + 1) * NP)], hi)

    def pair(p, _):
        for b in (0, 1):
            c = 2 * p + b
            pltpu.make_async_copy(edges.at[pl.ds(0, CHUNK)], ev.at[b], sems[b]).wait()
            process(b)

            @pl.when(c + 2 < NCHUNK)
            def _():
                off = (c + 2) * CHUNK
                pltpu.async_copy(edges.at[pl.ds(off, CHUNK)], ev.at[b], sems[b])

        return 0

    lax.fori_loop(0, NCHUNK // 2, pair, 0)

    for r in range(RPW):
        pltpu.sync_copy(accv.at[pl.ds(r * NP, NP)], aggt.at[RPW * wid + r])

    if with_counts:
        pltpu.sync_copy(edges.at[pl.ds(EPW * wid, EPW)], cev)

        @plsc.parallel_loop(0, NP // 16, unroll=8)
        def czero(i):
            cntv[pl.ds(i * 16, 16)] = zf

        ones = jnp.full((16,), 1.0, jnp.float32)

        @plsc.parallel_loop(0, EPW // 16, unroll=4)
        def cbody(g):
            _, dv = _unpack_edges(cev[pl.ds(g * 16, 16)])
            plsc.addupdate_scatter(cntv, [dv], ones)

        pltpu.sync_copy(cntv, cpart.at[wid])


def _make_sc_agg(with_counts):
    mesh = plsc.VectorSubcoreMesh(core_axis_name="c", subcore_axis_name="s")
    if with_counts:
        out_type = (jax.ShapeDtypeStruct((D, NP), jnp.float32),
                    jax.ShapeDtypeStruct((NW, NP), jnp.float32))
    else:
        out_type = jax.ShapeDtypeStruct((D, NP), jnp.float32)
    scratch = [
        pltpu.VMEM((PRPW * NP,), jnp.int32),
        pltpu.VMEM((RPW * NP,), jnp.float32),
        pltpu.VMEM((2, CHUNK), jnp.int32),
    ]
    if with_counts:
        scratch.append(pltpu.VMEM((NP,), jnp.float32))
        scratch.append(pltpu.VMEM((EPW,), jnp.int32))
    scratch += [pltpu.SemaphoreType.DMA] * 2
    return pl.kernel(
        functools.partial(_sc_agg_body, with_counts),
        mesh=mesh,
        out_type=out_type,
        scratch_types=scratch,
        compiler_params=pltpu.CompilerParams(needs_layout_passes=False),
    )


_sc_agg_counts = _make_sc_agg(True)
_sc_agg = _make_sc_agg(False)


def _pack_block(y):
    u = lax.bitcast_convert_type(y.astype(jnp.bfloat16),
                                 jnp.uint16).astype(jnp.uint32)
    ur = u.reshape(D // 2, 2, u.shape[-1])
    w = ur[:, 0, :] | (ur[:, 1, :] << 16)
    return lax.bitcast_convert_type(w, jnp.int32)


def _pre_body(wl_ref, x_ref, o_ref):
    dn = (((1,), (1,)), ((), ()))
    y = lax.dot_general(wl_ref[...], x_ref[...], dn,
                        preferred_element_type=jnp.float32)
    o_ref[...] = _pack_block(y)


def _pre(W_l, x):
    return pl.pallas_call(
        _pre_body,
        grid=(NP // BLK,),
        in_specs=[
            pl.BlockSpec((D, D), lambda i: (0, 0)),
            pl.BlockSpec((BLK, D), lambda i: (i, 0)),
        ],
        out_specs=pl.BlockSpec((D // 2, BLK), lambda i: (0, i)),
        out_shape=jax.ShapeDtypeStruct((D // 2, NP), jnp.int32),
    )(W_l, x)


def _eye():
    r = lax.broadcasted_iota(jnp.int32, (D, D), 0)
    c = lax.broadcasted_iota(jnp.int32, (D, D), 1)
    return (r == c).astype(jnp.float32)


def _z_body(x_ref, wr_ref, b_ref, z_ref):
    dn1 = (((1,), (1,)), ((), ()))
    z_ref[...] = lax.dot_general(x_ref[...], wr_ref[...], dn1,
                                 preferred_element_type=jnp.float32) + b_ref[...]


def _z(xin, W_r, b):
    return pl.pallas_call(
        _z_body,
        grid=(NP // BLK,),
        in_specs=[
            pl.BlockSpec((BLK, D), lambda i: (i, 0)),
            pl.BlockSpec((D, D), lambda i: (0, 0)),
            pl.BlockSpec((1, D), lambda i: (0, 0)),
        ],
        out_specs=pl.BlockSpec((BLK, D), lambda i: (i, 0)),
        out_shape=jax.ShapeDtypeStruct((N, D), jnp.float32),
    )(xin, W_r, b.reshape(1, D))


def _mid_body(agg_ref, cp_ref, z_ref, w2l_ref, h_ref, y2_ref):
    cnt = jnp.sum(cp_ref[...], axis=0, keepdims=True)
    inv = 1.0 / jnp.maximum(cnt, 1.0)
    t = agg_ref[...] * inv
    dn0 = (((0,), (0,)), ((), ()))
    dn1 = (((1,), (1,)), ((), ()))
    mean = lax.dot_general(t, _eye(), dn0, preferred_element_type=jnp.float32)
    h = jnp.maximum(mean + z_ref[...], 0.0)
    h_ref[...] = h
    y2 = lax.dot_general(w2l_ref[...], h, dn1,
                         preferred_element_type=jnp.float32)
    y2_ref[...] = _pack_block(y2)


def _mid(aggT, cpart, z1, W2_l):
    return pl.pallas_call(
        _mid_body,
        grid=(NP // BLK,),
        in_specs=[
            pl.BlockSpec((D, BLK), lambda i: (0, i)),
            pl.BlockSpec((NW, BLK), lambda i: (0, i)),
            pl.BlockSpec((BLK, D), lambda i: (i, 0)),
            pl.BlockSpec((D, D), lambda i: (0, 0)),
        ],
        out_specs=[
            pl.BlockSpec((BLK, D), lambda i: (i, 0)),
            pl.BlockSpec((D // 2, BLK), lambda i: (0, i)),
        ],
        out_shape=[
            jax.ShapeDtypeStruct((N, D), jnp.float32),
            jax.ShapeDtypeStruct((D // 2, NP), jnp.int32),
        ],
    )(aggT, cpart, z1, W2_l)


def _post_body(agg_ref, cp_ref, z_ref, o_ref):
    cnt = jnp.sum(cp_ref[...], axis=0, keepdims=True)
    inv = 1.0 / jnp.maximum(cnt, 1.0)
    t = agg_ref[...] * inv
    dn0 = (((0,), (0,)), ((), ()))
    mean = lax.dot_general(t, _eye(), dn0, preferred_element_type=jnp.float32)
    o_ref[...] = jnp.maximum(mean + z_ref[...], 0.0)


def _post(aggT, cpart, z2):
    return pl.pallas_call(
        _post_body,
        grid=(NP // BLK,),
        in_specs=[
            pl.BlockSpec((D, BLK), lambda i: (0, i)),
            pl.BlockSpec((NW, BLK), lambda i: (0, i)),
            pl.BlockSpec((BLK, D), lambda i: (i, 0)),
        ],
        out_specs=pl.BlockSpec((BLK, D), lambda i: (i, 0)),
        out_shape=jax.ShapeDtypeStruct((N, D), jnp.float32),
    )(aggT, cpart, z2)


def kernel(x, edge_index, W1_l, b1_l, W1_r, W2_l, b2_l, W2_r):
    src = edge_index[0].astype(jnp.int32)
    dst = edge_index[1].astype(jnp.int32)
    edges = src | (dst << 16)
    y1p = _pre(W1_l, x)
    agg1T, cpart = _sc_agg_counts(y1p, edges)
    z1 = _z(x, W1_r, b1_l)
    h1, y2p = _mid(agg1T, cpart, z1, W2_l)
    agg2T = _sc_agg(y2p, edges)
    z2 = _z(h1, W2_r, b2_l)
    return _post(agg2T, cpart, z2)


# speedup vs baseline: 1.0133x; 1.0062x over previous
import functools

import jax
import jax.numpy as jnp
from jax import lax
from jax.experimental import pallas as pl
from jax.experimental.pallas import tpu as pltpu
from jax.experimental.pallas import tpu_sc as plsc

N = 10000
NP = 10240
E = 320000
D = 128
NC = 2
NS = 16
NW = NC * NS
RPW = D // NW
PRPW = RPW // 2
EPW = E // NW
CHUNK = 3200
NCHUNK = E // CHUNK
GROUPS = CHUNK // 16
BLK = 1024


def _unpack_edges(ev):
    e16 = plsc.bitcast(ev, jnp.int16)
    sv, dv = plsc.unpack(e16, format=plsc.PackFormat.INTERLEAVED,
                         preferred_element_type=jnp.int32)
    return sv, dv


def _sc_agg_body(with_counts, *refs):
    if with_counts:
        (yt, edges, aggt, cpart,
         yv, accv, ev, cntv, cev,
         sem0, sem1) = refs
    else:
        (yt, edges, aggt,
         yv, accv, ev,
         sem0, sem1) = refs
    sems = (sem0, sem1)
    wid = lax.axis_index("s") * NC + lax.axis_index("c")

    for b in (0, 1):
        pltpu.async_copy(edges.at[pl.ds(b * CHUNK, CHUNK)], ev.at[b], sems[b])

    for r in range(PRPW):
        pltpu.sync_copy(yt.at[PRPW * wid + r], yv.at[pl.ds(r * NP, NP)])

    zf = jnp.zeros((16,), jnp.float32)

    @plsc.parallel_loop(0, (RPW * NP) // 16, unroll=8)
    def zbody(i):
        accv[pl.ds(i * 16, 16)] = zf

    def process(b):
        @plsc.parallel_loop(0, GROUPS, unroll=4)
        def gbody(g):
            sv, dv = _unpack_edges(ev[b, pl.ds(g * 16, 16)])
            for r in range(PRPW):
                pv = plsc.load_gather(yv, [sv + (r * NP)])
                lo, hi = plsc.unpack(plsc.bitcast(pv, jnp.bfloat16),
                                     format=plsc.PackFormat.INTERLEAVED)
                plsc.addupdate_scatter(accv, [dv + (2 * r * NP)], lo)
                plsc.addupdate_scatter(accv, [dv + ((2 * r + 1) * NP)], hi)

    def pair(p, _):
        for b in (0, 1):
            c = 2 * p + b
            pltpu.make_async_copy(edges.at[pl.ds(0, CHUNK)], ev.at[b], sems[b]).wait()
            process(b)

            @pl.when(c + 2 < NCHUNK)
            def _():
                off = (c + 2) * CHUNK
                pltpu.async_copy(edges.at[pl.ds(off, CHUNK)], ev.at[b], sems[b])

        return 0

    lax.fori_loop(0, NCHUNK // 2, pair, 0)

    for r in range(RPW):
        pltpu.sync_copy(accv.at[pl.ds(r * NP, NP)], aggt.at[RPW * wid + r])

    if with_counts:
        pltpu.sync_copy(edges.at[pl.ds(EPW * wid, EPW)], cev)

        @plsc.parallel_loop(0, NP // 16, unroll=8)
        def czero(i):
            cntv[pl.ds(i * 16, 16)] = zf

        ones = jnp.full((16,), 1.0, jnp.float32)

        @plsc.parallel_loop(0, EPW // 16, unroll=4)
        def cbody(g):
            _, dv = _unpack_edges(cev[pl.ds(g * 16, 16)])
            plsc.addupdate_scatter(cntv, [dv], ones)

        pltpu.sync_copy(cntv, cpart.at[wid])


def _make_sc_agg(with_counts):
    mesh = plsc.VectorSubcoreMesh(core_axis_name="c", subcore_axis_name="s")
    if with_counts:
        out_type = (jax.ShapeDtypeStruct((D, NP), jnp.float32),
                    jax.ShapeDtypeStruct((NW, NP), jnp.float32))
    else:
        out_type = jax.ShapeDtypeStruct((D, NP), jnp.float32)
    scratch = [
        pltpu.VMEM((PRPW * NP,), jnp.int32),
        pltpu.VMEM((RPW * NP,), jnp.float32),
        pltpu.VMEM((2, CHUNK), jnp.int32),
    ]
    if with_counts:
        scratch.append(pltpu.VMEM((NP,), jnp.float32))
        scratch.append(pltpu.VMEM((EPW,), jnp.int32))
    scratch += [pltpu.SemaphoreType.DMA] * 2
    return pl.kernel(
        functools.partial(_sc_agg_body, with_counts),
        mesh=mesh,
        out_type=out_type,
        scratch_types=scratch,
        compiler_params=pltpu.CompilerParams(needs_layout_passes=False),
    )


_sc_agg_counts = _make_sc_agg(True)
_sc_agg = _make_sc_agg(False)


def _pack_block(y):
    u = lax.bitcast_convert_type(y.astype(jnp.bfloat16),
                                 jnp.uint16).astype(jnp.uint32)
    ur = u.reshape(D // 2, 2, u.shape[-1])
    w = ur[:, 0, :] | (ur[:, 1, :] << 16)
    return lax.bitcast_convert_type(w, jnp.int32)


EBLK = E // (NP // BLK)


def _pre_body(wl_ref, x_ref, e_ref, o_ref, eo_ref):
    dn = (((1,), (1,)), ((), ()))
    y = lax.dot_general(wl_ref[...], x_ref[...], dn,
                        preferred_element_type=jnp.float32)
    o_ref[...] = _pack_block(y)
    ei = e_ref[...]
    eo_ref[...] = ei[0:1, :] | (ei[1:2, :] << 16)


def _pre(W_l, x, edge_index):
    return pl.pallas_call(
        _pre_body,
        grid=(NP // BLK,),
        in_specs=[
            pl.BlockSpec((D, D), lambda i: (0, 0)),
            pl.BlockSpec((BLK, D), lambda i: (i, 0)),
            pl.BlockSpec((2, EBLK), lambda i: (0, i)),
        ],
        out_specs=[
            pl.BlockSpec((D // 2, BLK), lambda i: (0, i)),
            pl.BlockSpec((1, EBLK), lambda i: (0, i)),
        ],
        out_shape=[
            jax.ShapeDtypeStruct((D // 2, NP), jnp.int32),
            jax.ShapeDtypeStruct((1, E), jnp.int32),
        ],
    )(W_l, x, edge_index)


def _eye():
    r = lax.broadcasted_iota(jnp.int32, (D, D), 0)
    c = lax.broadcasted_iota(jnp.int32, (D, D), 1)
    return (r == c).astype(jnp.float32)


def _mid_body(agg_ref, cp_ref, x_ref, wr_ref, b_ref, w2l_ref, h_ref, y2_ref):
    cnt = jnp.sum(cp_ref[...], axis=0, keepdims=True)
    inv = 1.0 / jnp.maximum(cnt, 1.0)
    t = agg_ref[...] * inv
    dn0 = (((0,), (0,)), ((), ()))
    dn1 = (((1,), (1,)), ((), ()))
    mean = lax.dot_general(t, _eye(), dn0, preferred_element_type=jnp.float32)
    h = mean + lax.dot_general(x_ref[...], wr_ref[...], dn1,
                               preferred_element_type=jnp.float32)
    h = jnp.maximum(h + b_ref[...], 0.0)
    h_ref[...] = h
    y2 = lax.dot_general(w2l_ref[...], h, dn1,
                         preferred_element_type=jnp.float32)
    y2_ref[...] = _pack_block(y2)


def _mid(aggT, cpart, x, W_r, b, W2_l):
    return pl.pallas_call(
        _mid_body,
        grid=(NP // BLK,),
        in_specs=[
            pl.BlockSpec((D, BLK), lambda i: (0, i)),
            pl.BlockSpec((NW, BLK), lambda i: (0, i)),
            pl.BlockSpec((BLK, D), lambda i: (i, 0)),
            pl.BlockSpec((D, D), lambda i: (0, 0)),
            pl.BlockSpec((1, D), lambda i: (0, 0)),
            pl.BlockSpec((D, D), lambda i: (0, 0)),
        ],
        out_specs=[
            pl.BlockSpec((BLK, D), lambda i: (i, 0)),
            pl.BlockSpec((D // 2, BLK), lambda i: (0, i)),
        ],
        out_shape=[
            jax.ShapeDtypeStruct((N, D), jnp.float32),
            jax.ShapeDtypeStruct((D // 2, NP), jnp.int32),
        ],
    )(aggT, cpart, x, W_r, b.reshape(1, D), W2_l)


def _post_body(agg_ref, cp_ref, x_ref, wr_ref, b_ref, o_ref):
    cnt = jnp.sum(cp_ref[...], axis=0, keepdims=True)
    inv = 1.0 / jnp.maximum(cnt, 1.0)
    t = agg_ref[...] * inv
    dn0 = (((0,), (0,)), ((), ()))
    dn1 = (((1,), (1,)), ((), ()))
    mean = lax.dot_general(t, _eye(), dn0, preferred_element_type=jnp.float32)
    h = mean + lax.dot_general(x_ref[...], wr_ref[...], dn1,
                               preferred_element_type=jnp.float32)
    o_ref[...] = jnp.maximum(h + b_ref[...], 0.0)


def _post(aggT, cpart, h1, W_r, b):
    return pl.pallas_call(
        _post_body,
        grid=(NP // BLK,),
        in_specs=[
            pl.BlockSpec((D, BLK), lambda i: (0, i)),
            pl.BlockSpec((NW, BLK), lambda i: (0, i)),
            pl.BlockSpec((BLK, D), lambda i: (i, 0)),
            pl.BlockSpec((D, D), lambda i: (0, 0)),
            pl.BlockSpec((1, D), lambda i: (0, 0)),
        ],
        out_specs=pl.BlockSpec((BLK, D), lambda i: (i, 0)),
        out_shape=jax.ShapeDtypeStruct((N, D), jnp.float32),
    )(aggT, cpart, h1, W_r, b.reshape(1, D))


def kernel(x, edge_index, W1_l, b1_l, W1_r, W2_l, b2_l, W2_r):
    y1p, edges2d = _pre(W1_l, x, edge_index.astype(jnp.int32))
    edges = edges2d.reshape(E)
    agg1T, cpart = _sc_agg_counts(y1p, edges)
    h1, y2p = _mid(agg1T, cpart, x, W1_r, b1_l, W2_l)
    agg2T = _sc_agg(y2p, edges)
    return _post(agg2T, cpart, h1, W2_r, b2_l)


# speedup vs baseline: 1.0250x; 1.0116x over previous
import functools

import jax
import jax.numpy as jnp
from jax import lax
from jax.experimental import pallas as pl
from jax.experimental.pallas import tpu as pltpu
from jax.experimental.pallas import tpu_sc as plsc

N = 10000
NP = 10240
E = 320000
D = 128
NC = 2
NS = 16
NW = NC * NS
RPW = D // NW
PRPW = RPW // 2
EPW = E // NW
CHUNK = 3200
NCHUNK = E // CHUNK
GROUPS = CHUNK // 16
BLK = 1024


def _unpack_edges(ev):
    e16 = plsc.bitcast(ev, jnp.int16)
    sv, dv = plsc.unpack(e16, format=plsc.PackFormat.INTERLEAVED,
                         preferred_element_type=jnp.int32)
    return sv, dv


def _sc_agg_body(with_counts, *refs):
    if with_counts:
        (yt, edges, aggt, cpart,
         yv, accv, ev, cntv, cev,
         sem0, sem1, sem_y) = refs
    else:
        (yt, edges, aggt,
         yv, accv, ev,
         sem0, sem1, sem_y) = refs
    sems = (sem0, sem1)
    wid = lax.axis_index("s") * NC + lax.axis_index("c")

    for b in (0, 1):
        pltpu.async_copy(edges.at[pl.ds(b * CHUNK, CHUNK)], ev.at[b], sems[b])

    for r in range(PRPW):
        pltpu.async_copy(yt.at[PRPW * wid + r], yv.at[pl.ds(r * NP, NP)], sem_y)

    zf = jnp.zeros((16,), jnp.float32)

    @plsc.parallel_loop(0, (RPW * NP) // 16, unroll=8)
    def zbody(i):
        accv[pl.ds(i * 16, 16)] = zf

    for r in range(PRPW):
        pltpu.make_async_copy(yt.at[PRPW * wid + r],
                              yv.at[pl.ds(r * NP, NP)], sem_y).wait()

    def process(b):
        @plsc.parallel_loop(0, GROUPS, unroll=4)
        def gbody(g):
            sv, dv = _unpack_edges(ev[b, pl.ds(g * 16, 16)])
            for r in range(PRPW):
                pv = plsc.load_gather(yv, [sv + (r * NP)])
                lo, hi = plsc.unpack(plsc.bitcast(pv, jnp.bfloat16),
                                     format=plsc.PackFormat.INTERLEAVED)
                plsc.addupdate_scatter(accv, [dv + (2 * r * NP)], lo)
                plsc.addupdate_scatter(accv, [dv + ((2 * r + 1) * NP)], hi)

    def pair(p, _):
        for b in (0, 1):
            c = 2 * p + b
            pltpu.make_async_copy(edges.at[pl.ds(0, CHUNK)], ev.at[b], sems[b]).wait()
            process(b)

            @pl.when(c + 2 < NCHUNK)
            def _():
                off = (c + 2) * CHUNK
                pltpu.async_copy(edges.at[pl.ds(off, CHUNK)], ev.at[b], sems[b])

        return 0

    lax.fori_loop(0, NCHUNK // 2, pair, 0)

    for r in range(RPW):
        pltpu.async_copy(accv.at[pl.ds(r * NP, NP)], aggt.at[RPW * wid + r],
                         sem_y)

    if with_counts:
        pltpu.sync_copy(edges.at[pl.ds(EPW * wid, EPW)], cev)

        @plsc.parallel_loop(0, NP // 16, unroll=8)
        def czero(i):
            cntv[pl.ds(i * 16, 16)] = zf

        ones = jnp.full((16,), 1.0, jnp.float32)

        @plsc.parallel_loop(0, EPW // 16, unroll=4)
        def cbody(g):
            _, dv = _unpack_edges(cev[pl.ds(g * 16, 16)])
            plsc.addupdate_scatter(cntv, [dv], ones)

        pltpu.sync_copy(cntv, cpart.at[wid])

    for r in range(RPW):
        pltpu.make_async_copy(accv.at[pl.ds(r * NP, NP)],
                              aggt.at[RPW * wid + r], sem_y).wait()


def _make_sc_agg(with_counts):
    mesh = plsc.VectorSubcoreMesh(core_axis_name="c", subcore_axis_name="s")
    if with_counts:
        out_type = (jax.ShapeDtypeStruct((D, NP), jnp.float32),
                    jax.ShapeDtypeStruct((NW, NP), jnp.float32))
    else:
        out_type = jax.ShapeDtypeStruct((D, NP), jnp.float32)
    scratch = [
        pltpu.VMEM((PRPW * NP,), jnp.int32),
        pltpu.VMEM((RPW * NP,), jnp.float32),
        pltpu.VMEM((2, CHUNK), jnp.int32),
    ]
    if with_counts:
        scratch.append(pltpu.VMEM((NP,), jnp.float32))
        scratch.append(pltpu.VMEM((EPW,), jnp.int32))
    scratch += [pltpu.SemaphoreType.DMA] * 3
    return pl.kernel(
        functools.partial(_sc_agg_body, with_counts),
        mesh=mesh,
        out_type=out_type,
        scratch_types=scratch,
        compiler_params=pltpu.CompilerParams(needs_layout_passes=False),
    )


_sc_agg_counts = _make_sc_agg(True)
_sc_agg = _make_sc_agg(False)


def _pack_block(y):
    u = lax.bitcast_convert_type(y.astype(jnp.bfloat16),
                                 jnp.uint16).astype(jnp.uint32)
    ur = u.reshape(D // 2, 2, u.shape[-1])
    w = ur[:, 0, :] | (ur[:, 1, :] << 16)
    return lax.bitcast_convert_type(w, jnp.int32)


EBLK = E // (NP // BLK)


def _pre_body(wl_ref, x_ref, e_ref, o_ref, eo_ref):
    dn = (((1,), (1,)), ((), ()))
    y = lax.dot_general(wl_ref[...], x_ref[...], dn,
                        preferred_element_type=jnp.float32)
    o_ref[...] = _pack_block(y)
    ei = e_ref[...]
    eo_ref[...] = ei[0:1, :] | (ei[1:2, :] << 16)


def _pre(W_l, x, edge_index):
    return pl.pallas_call(
        _pre_body,
        grid=(NP // BLK,),
        in_specs=[
            pl.BlockSpec((D, D), lambda i: (0, 0)),
            pl.BlockSpec((BLK, D), lambda i: (i, 0)),
            pl.BlockSpec((2, EBLK), lambda i: (0, i)),
        ],
        out_specs=[
            pl.BlockSpec((D // 2, BLK), lambda i: (0, i)),
            pl.BlockSpec((1, EBLK), lambda i: (0, i)),
        ],
        out_shape=[
            jax.ShapeDtypeStruct((D // 2, NP), jnp.int32),
            jax.ShapeDtypeStruct((1, E), jnp.int32),
        ],
    )(W_l, x, edge_index)


def _eye():
    r = lax.broadcasted_iota(jnp.int32, (D, D), 0)
    c = lax.broadcasted_iota(jnp.int32, (D, D), 1)
    return (r == c).astype(jnp.float32)


def _mid_body(agg_ref, cp_ref, x_ref, wr_ref, b_ref, w2l_ref, h_ref, y2_ref):
    cnt = jnp.sum(cp_ref[...], axis=0, keepdims=True)
    inv = 1.0 / jnp.maximum(cnt, 1.0)
    t = agg_ref[...] * inv
    dn0 = (((0,), (0,)), ((), ()))
    dn1 = (((1,), (1,)), ((), ()))
    mean = lax.dot_general(t, _eye(), dn0, preferred_element_type=jnp.float32)
    h = mean + lax.dot_general(x_ref[...], wr_ref[...], dn1,
                               preferred_element_type=jnp.float32)
    h = jnp.maximum(h + b_ref[...], 0.0)
    h_ref[...] = h
    y2 = lax.dot_general(w2l_ref[...], h, dn1,
                         preferred_element_type=jnp.float32)
    y2_ref[...] = _pack_block(y2)


def _mid(aggT, cpart, x, W_r, b, W2_l):
    return pl.pallas_call(
        _mid_body,
        grid=(NP // BLK,),
        in_specs=[
            pl.BlockSpec((D, BLK), lambda i: (0, i)),
            pl.BlockSpec((NW, BLK), lambda i: (0, i)),
            pl.BlockSpec((BLK, D), lambda i: (i, 0)),
            pl.BlockSpec((D, D), lambda i: (0, 0)),
            pl.BlockSpec((1, D), lambda i: (0, 0)),
            pl.BlockSpec((D, D), lambda i: (0, 0)),
        ],
        out_specs=[
            pl.BlockSpec((BLK, D), lambda i: (i, 0)),
            pl.BlockSpec((D // 2, BLK), lambda i: (0, i)),
        ],
        out_shape=[
            jax.ShapeDtypeStruct((N, D), jnp.float32),
            jax.ShapeDtypeStruct((D // 2, NP), jnp.int32),
        ],
    )(aggT, cpart, x, W_r, b.reshape(1, D), W2_l)


def _post_body(agg_ref, cp_ref, x_ref, wr_ref, b_ref, o_ref):
    cnt = jnp.sum(cp_ref[...], axis=0, keepdims=True)
    inv = 1.0 / jnp.maximum(cnt, 1.0)
    t = agg_ref[...] * inv
    dn0 = (((0,), (0,)), ((), ()))
    dn1 = (((1,), (1,)), ((), ()))
    mean = lax.dot_general(t, _eye(), dn0, preferred_element_type=jnp.float32)
    h = mean + lax.dot_general(x_ref[...], wr_ref[...], dn1,
                               preferred_element_type=jnp.float32)
    o_ref[...] = jnp.maximum(h + b_ref[...], 0.0)


def _post(aggT, cpart, h1, W_r, b):
    return pl.pallas_call(
        _post_body,
        grid=(NP // BLK,),
        in_specs=[
            pl.BlockSpec((D, BLK), lambda i: (0, i)),
            pl.BlockSpec((NW, BLK), lambda i: (0, i)),
            pl.BlockSpec((BLK, D), lambda i: (i, 0)),
            pl.BlockSpec((D, D), lambda i: (0, 0)),
            pl.BlockSpec((1, D), lambda i: (0, 0)),
        ],
        out_specs=pl.BlockSpec((BLK, D), lambda i: (i, 0)),
        out_shape=jax.ShapeDtypeStruct((N, D), jnp.float32),
    )(aggT, cpart, h1, W_r, b.reshape(1, D))


def kernel(x, edge_index, W1_l, b1_l, W1_r, W2_l, b2_l, W2_r):
    y1p, edges2d = _pre(W1_l, x, edge_index.astype(jnp.int32))
    edges = edges2d.reshape(E)
    agg1T, cpart = _sc_agg_counts(y1p, edges)
    h1, y2p = _mid(agg1T, cpart, x, W1_r, b1_l, W2_l)
    agg2T = _sc_agg(y2p, edges)
    return _post(agg2T, cpart, h1, W2_r, b2_l)


# speedup vs baseline: 1.0254x; 1.0004x over previous
import functools

import jax
import jax.numpy as jnp
from jax import lax
from jax.experimental import pallas as pl
from jax.experimental.pallas import tpu as pltpu
from jax.experimental.pallas import tpu_sc as plsc

N = 10000
NP = 10240
E = 320000
D = 128
NC = 2
NS = 16
NW = NC * NS
RPW = D // NW
PRPW = RPW // 2
EPW = E // NW
CHUNK = 3200
NCHUNK = E // CHUNK
GROUPS = CHUNK // 16
BLK = 1024


def _unpack_edges(ev):
    e16 = plsc.bitcast(ev, jnp.int16)
    sv, dv = plsc.unpack(e16, format=plsc.PackFormat.INTERLEAVED,
                         preferred_element_type=jnp.int32)
    return sv, dv


def _sc_agg_body(with_counts, *refs):
    if with_counts:
        (yt, edges, aggt, cpart,
         yv, accv, ev, cntv, cev,
         sem0, sem1, sem_y) = refs
    else:
        (yt, edges, aggt,
         yv, accv, ev,
         sem0, sem1, sem_y) = refs
    sems = (sem0, sem1)
    wid = lax.axis_index("s") * NC + lax.axis_index("c")

    for b in (0, 1):
        pltpu.async_copy(edges.at[pl.ds(b * CHUNK, CHUNK)], ev.at[b], sems[b])

    for r in range(PRPW):
        pltpu.async_copy(yt.at[PRPW * wid + r], yv.at[pl.ds(r * NP, NP)], sem_y)

    zf = jnp.zeros((16,), jnp.float32)

    @plsc.parallel_loop(0, (RPW * NP) // 16, unroll=8)
    def zbody(i):
        accv[pl.ds(i * 16, 16)] = zf

    for r in range(PRPW):
        pltpu.make_async_copy(yt.at[PRPW * wid + r],
                              yv.at[pl.ds(r * NP, NP)], sem_y).wait()

    def process(b):
        @plsc.parallel_loop(0, GROUPS, unroll=5)
        def gbody(g):
            sv, dv = _unpack_edges(ev[b, pl.ds(g * 16, 16)])
            for r in range(PRPW):
                pv = plsc.load_gather(yv, [sv + (r * NP)])
                lo, hi = plsc.unpack(plsc.bitcast(pv, jnp.bfloat16),
                                     format=plsc.PackFormat.INTERLEAVED)
                plsc.addupdate_scatter(accv, [dv + (2 * r * NP)], lo)
                plsc.addupdate_scatter(accv, [dv + ((2 * r + 1) * NP)], hi)

    def pair(p, _):
        for b in (0, 1):
            c = 2 * p + b
            pltpu.make_async_copy(edges.at[pl.ds(0, CHUNK)], ev.at[b], sems[b]).wait()
            process(b)

            @pl.when(c + 2 < NCHUNK)
            def _():
                off = (c + 2) * CHUNK
                pltpu.async_copy(edges.at[pl.ds(off, CHUNK)], ev.at[b], sems[b])

        return 0

    lax.fori_loop(0, NCHUNK // 2, pair, 0)

    for r in range(RPW):
        pltpu.async_copy(accv.at[pl.ds(r * NP, NP)], aggt.at[RPW * wid + r],
                         sem_y)

    if with_counts:
        pltpu.sync_copy(edges.at[pl.ds(EPW * wid, EPW)], cev)

        @plsc.parallel_loop(0, NP // 16, unroll=8)
        def czero(i):
            cntv[pl.ds(i * 16, 16)] = zf

        ones = jnp.full((16,), 1.0, jnp.float32)

        @plsc.parallel_loop(0, EPW // 16, unroll=4)
        def cbody(g):
            _, dv = _unpack_edges(cev[pl.ds(g * 16, 16)])
            plsc.addupdate_scatter(cntv, [dv], ones)

        pltpu.sync_copy(cntv, cpart.at[wid])

    for r in range(RPW):
        pltpu.make_async_copy(accv.at[pl.ds(r * NP, NP)],
                              aggt.at[RPW * wid + r], sem_y).wait()


def _make_sc_agg(with_counts):
    mesh = plsc.VectorSubcoreMesh(core_axis_name="c", subcore_axis_name="s")
    if with_counts:
        out_type = (jax.ShapeDtypeStruct((D, NP), jnp.float32),
                    jax.ShapeDtypeStruct((NW, NP), jnp.float32))
    else:
        out_type = jax.ShapeDtypeStruct((D, NP), jnp.float32)
    scratch = [
        pltpu.VMEM((PRPW * NP,), jnp.int32),
        pltpu.VMEM((RPW * NP,), jnp.float32),
        pltpu.VMEM((2, CHUNK), jnp.int32),
    ]
    if with_counts:
        scratch.append(pltpu.VMEM((NP,), jnp.float32))
        scratch.append(pltpu.VMEM((EPW,), jnp.int32))
    scratch += [pltpu.SemaphoreType.DMA] * 3
    return pl.kernel(
        functools.partial(_sc_agg_body, with_counts),
        mesh=mesh,
        out_type=out_type,
        scratch_types=scratch,
        compiler_params=pltpu.CompilerParams(needs_layout_passes=False),
    )


_sc_agg_counts = _make_sc_agg(True)
_sc_agg = _make_sc_agg(False)


def _pack_block(y):
    u = lax.bitcast_convert_type(y.astype(jnp.bfloat16),
                                 jnp.uint16).astype(jnp.uint32)
    ur = u.reshape(D // 2, 2, u.shape[-1])
    w = ur[:, 0, :] | (ur[:, 1, :] << 16)
    return lax.bitcast_convert_type(w, jnp.int32)


EBLK = E // (NP // BLK)


def _pre_body(wl_ref, x_ref, e_ref, o_ref, eo_ref):
    dn = (((1,), (1,)), ((), ()))
    y = lax.dot_general(wl_ref[...], x_ref[...], dn,
                        preferred_element_type=jnp.float32)
    o_ref[...] = _pack_block(y)
    ei = e_ref[...]
    eo_ref[...] = ei[0:1, :] | (ei[1:2, :] << 16)


def _pre(W_l, x, edge_index):
    return pl.pallas_call(
        _pre_body,
        grid=(NP // BLK,),
        in_specs=[
            pl.BlockSpec((D, D), lambda i: (0, 0)),
            pl.BlockSpec((BLK, D), lambda i: (i, 0)),
            pl.BlockSpec((2, EBLK), lambda i: (0, i)),
        ],
        out_specs=[
            pl.BlockSpec((D // 2, BLK), lambda i: (0, i)),
            pl.BlockSpec((1, EBLK), lambda i: (0, i)),
        ],
        out_shape=[
            jax.ShapeDtypeStruct((D // 2, NP), jnp.int32),
            jax.ShapeDtypeStruct((1, E), jnp.int32),
        ],
    )(W_l, x, edge_index)


def _eye():
    r = lax.broadcasted_iota(jnp.int32, (D, D), 0)
    c = lax.broadcasted_iota(jnp.int32, (D, D), 1)
    return (r == c).astype(jnp.float32)


def _mid_body(agg_ref, cp_ref, x_ref, wr_ref, b_ref, w2l_ref, h_ref, y2_ref):
    cnt = jnp.sum(cp_ref[...], axis=0, keepdims=True)
    inv = 1.0 / jnp.maximum(cnt, 1.0)
    t = agg_ref[...] * inv
    dn0 = (((0,), (0,)), ((), ()))
    dn1 = (((1,), (1,)), ((), ()))
    mean = lax.dot_general(t, _eye(), dn0, preferred_element_type=jnp.float32)
    h = mean + lax.dot_general(x_ref[...], wr_ref[...], dn1,
                               preferred_element_type=jnp.float32)
    h = jnp.maximum(h + b_ref[...], 0.0)
    h_ref[...] = h
    y2 = lax.dot_general(w2l_ref[...], h, dn1,
                         preferred_element_type=jnp.float32)
    y2_ref[...] = _pack_block(y2)


def _mid(aggT, cpart, x, W_r, b, W2_l):
    return pl.pallas_call(
        _mid_body,
        grid=(NP // BLK,),
        in_specs=[
            pl.BlockSpec((D, BLK), lambda i: (0, i)),
            pl.BlockSpec((NW, BLK), lambda i: (0, i)),
            pl.BlockSpec((BLK, D), lambda i: (i, 0)),
            pl.BlockSpec((D, D), lambda i: (0, 0)),
            pl.BlockSpec((1, D), lambda i: (0, 0)),
            pl.BlockSpec((D, D), lambda i: (0, 0)),
        ],
        out_specs=[
            pl.BlockSpec((BLK, D), lambda i: (i, 0)),
            pl.BlockSpec((D // 2, BLK), lambda i: (0, i)),
        ],
        out_shape=[
            jax.ShapeDtypeStruct((N, D), jnp.float32),
            jax.ShapeDtypeStruct((D // 2, NP), jnp.int32),
        ],
    )(aggT, cpart, x, W_r, b.reshape(1, D), W2_l)


def _post_body(agg_ref, cp_ref, x_ref, wr_ref, b_ref, o_ref):
    cnt = jnp.sum(cp_ref[...], axis=0, keepdims=True)
    inv = 1.0 / jnp.maximum(cnt, 1.0)
    t = agg_ref[...] * inv
    dn0 = (((0,), (0,)), ((), ()))
    dn1 = (((1,), (1,)), ((), ()))
    mean = lax.dot_general(t, _eye(), dn0, preferred_element_type=jnp.float32)
    h = mean + lax.dot_general(x_ref[...], wr_ref[...], dn1,
                               preferred_element_type=jnp.float32)
    o_ref[...] = jnp.maximum(h + b_ref[...], 0.0)


def _post(aggT, cpart, h1, W_r, b):
    return pl.pallas_call(
        _post_body,
        grid=(NP // BLK,),
        in_specs=[
            pl.BlockSpec((D, BLK), lambda i: (0, i)),
            pl.BlockSpec((NW, BLK), lambda i: (0, i)),
            pl.BlockSpec((BLK, D), lambda i: (i, 0)),
            pl.BlockSpec((D, D), lambda i: (0, 0)),
            pl.BlockSpec((1, D), lambda i: (0, 0)),
        ],
        out_specs=pl.BlockSpec((BLK, D), lambda i: (i, 0)),
        out_shape=jax.ShapeDtypeStruct((N, D), jnp.float32),
    )(aggT, cpart, h1, W_r, b.reshape(1, D))


def kernel(x, edge_index, W1_l, b1_l, W1_r, W2_l, b2_l, W2_r):
    y1p, edges2d = _pre(W1_l, x, edge_index.astype(jnp.int32))
    edges = edges2d.reshape(E)
    agg1T, cpart = _sc_agg_counts(y1p, edges)
    h1, y2p = _mid(agg1T, cpart, x, W1_r, b1_l, W2_l)
    agg2T = _sc_agg(y2p, edges)
    return _post(agg2T, cpart, h1, W2_r, b2_l)


# speedup vs baseline: 1.0257x; 1.0003x over previous
import functools

import jax
import jax.numpy as jnp
from jax import lax
from jax.experimental import pallas as pl
from jax.experimental.pallas import tpu as pltpu
from jax.experimental.pallas import tpu_sc as plsc

N = 10000
NP = 10240
E = 320000
D = 128
NC = 2
NS = 16
NW = NC * NS
RPW = D // NW
PRPW = RPW // 2
EPW = E // NW
CHUNK = 3200
NCHUNK = E // CHUNK
GROUPS = CHUNK // 16
BLK = 1024


def _unpack_edges(ev):
    e16 = plsc.bitcast(ev, jnp.int16)
    sv, dv = plsc.unpack(e16, format=plsc.PackFormat.INTERLEAVED,
                         preferred_element_type=jnp.int32)
    return sv, dv


def _sc_agg_body(with_counts, *refs):
    if with_counts:
        (yt, edges, aggt, cpart,
         yv, accv, ev, cntv, cev,
         sem0, sem1, sem_y) = refs
    else:
        (yt, edges, aggt,
         yv, accv, ev,
         sem0, sem1, sem_y) = refs
    sems = (sem0, sem1)
    wid = lax.axis_index("s") * NC + lax.axis_index("c")

    for b in (0, 1):
        pltpu.async_copy(edges.at[pl.ds(b * CHUNK, CHUNK)], ev.at[b], sems[b])

    for r in range(PRPW):
        pltpu.async_copy(yt.at[PRPW * wid + r], yv.at[pl.ds(r * NP, NP)], sem_y)

    zf = jnp.zeros((16,), jnp.float32)

    @plsc.parallel_loop(0, (RPW * NP) // 16, unroll=8)
    def zbody(i):
        accv[pl.ds(i * 16, 16)] = zf

    for r in range(PRPW):
        pltpu.make_async_copy(yt.at[PRPW * wid + r],
                              yv.at[pl.ds(r * NP, NP)], sem_y).wait()

    def process(b):
        @plsc.parallel_loop(0, GROUPS, unroll=4)
        def gbody(g):
            sv, dv = _unpack_edges(ev[b, pl.ds(g * 16, 16)])
            for r in range(PRPW):
                pv = plsc.load_gather(yv, [sv + (r * NP)])
                lo, hi = plsc.unpack(plsc.bitcast(pv, jnp.bfloat16),
                                     format=plsc.PackFormat.INTERLEAVED)
                plsc.addupdate_scatter(accv, [dv + (2 * r * NP)], lo)
                plsc.addupdate_scatter(accv, [dv + ((2 * r + 1) * NP)], hi)

    def pair(p, _):
        for b in (0, 1):
            c = 2 * p + b
            pltpu.make_async_copy(edges.at[pl.ds(0, CHUNK)], ev.at[b], sems[b]).wait()
            process(b)

            @pl.when(c + 2 < NCHUNK)
            def _():
                off = (c + 2) * CHUNK
                pltpu.async_copy(edges.at[pl.ds(off, CHUNK)], ev.at[b], sems[b])

        return 0

    lax.fori_loop(0, NCHUNK // 2, pair, 0)

    for r in range(RPW):
        pltpu.async_copy(accv.at[pl.ds(r * NP, NP)], aggt.at[RPW * wid + r],
                         sem_y)

    if with_counts:
        pltpu.sync_copy(edges.at[pl.ds(EPW * wid, EPW)], cev)

        @plsc.parallel_loop(0, NP // 16, unroll=8)
        def czero(i):
            cntv[pl.ds(i * 16, 16)] = zf

        ones = jnp.full((16,), 1.0, jnp.float32)

        @plsc.parallel_loop(0, EPW // 16, unroll=4)
        def cbody(g):
            _, dv = _unpack_edges(cev[pl.ds(g * 16, 16)])
            plsc.addupdate_scatter(cntv, [dv], ones)

        pltpu.sync_copy(cntv, cpart.at[wid])

    for r in range(RPW):
        pltpu.make_async_copy(accv.at[pl.ds(r * NP, NP)],
                              aggt.at[RPW * wid + r], sem_y).wait()


def _make_sc_agg(with_counts):
    mesh = plsc.VectorSubcoreMesh(core_axis_name="c", subcore_axis_name="s")
    if with_counts:
        out_type = (jax.ShapeDtypeStruct((D, NP), jnp.float32),
                    jax.ShapeDtypeStruct((NW, NP), jnp.float32))
    else:
        out_type = jax.ShapeDtypeStruct((D, NP), jnp.float32)
    scratch = [
        pltpu.VMEM((PRPW * NP,), jnp.int32),
        pltpu.VMEM((RPW * NP,), jnp.float32),
        pltpu.VMEM((2, CHUNK), jnp.int32),
    ]
    if with_counts:
        scratch.append(pltpu.VMEM((NP,), jnp.float32))
        scratch.append(pltpu.VMEM((EPW,), jnp.int32))
    scratch += [pltpu.SemaphoreType.DMA] * 3
    return pl.kernel(
        functools.partial(_sc_agg_body, with_counts),
        mesh=mesh,
        out_type=out_type,
        scratch_types=scratch,
        compiler_params=pltpu.CompilerParams(needs_layout_passes=False),
    )


_sc_agg_counts = _make_sc_agg(True)
_sc_agg = _make_sc_agg(False)


def _pack_block(y):
    u = lax.bitcast_convert_type(y.astype(jnp.bfloat16),
                                 jnp.uint16).astype(jnp.uint32)
    ur = u.reshape(D // 2, 2, u.shape[-1])
    w = ur[:, 0, :] | (ur[:, 1, :] << 16)
    return lax.bitcast_convert_type(w, jnp.int32)


EBLK = E // (NP // BLK)


def _pre_body(wl_ref, x_ref, e_ref, o_ref, eo_ref):
    dn = (((1,), (1,)), ((), ()))
    y = lax.dot_general(wl_ref[...], x_ref[...], dn,
                        preferred_element_type=jnp.float32)
    o_ref[...] = _pack_block(y)
    ei = e_ref[...]
    eo_ref[...] = ei[0:1, :] | (ei[1:2, :] << 16)


def _pre(W_l, x, edge_index):
    return pl.pallas_call(
        _pre_body,
        grid=(NP // BLK,),
        in_specs=[
            pl.BlockSpec((D, D), lambda i: (0, 0)),
            pl.BlockSpec((BLK, D), lambda i: (i, 0)),
            pl.BlockSpec((2, EBLK), lambda i: (0, i)),
        ],
        out_specs=[
            pl.BlockSpec((D // 2, BLK), lambda i: (0, i)),
            pl.BlockSpec((1, EBLK), lambda i: (0, i)),
        ],
        out_shape=[
            jax.ShapeDtypeStruct((D // 2, NP), jnp.int32),
            jax.ShapeDtypeStruct((1, E), jnp.int32),
        ],
    )(W_l, x, edge_index)


def _eye():
    r = lax.broadcasted_iota(jnp.int32, (D, D), 0)
    c = lax.broadcasted_iota(jnp.int32, (D, D), 1)
    return (r == c).astype(jnp.float32)


def _mid_body(agg_ref, cp_ref, x_ref, wr_ref, b_ref, w2l_ref, h_ref, y2_ref):
    cnt = jnp.sum(cp_ref[...], axis=0, keepdims=True)
    inv = 1.0 / jnp.maximum(cnt, 1.0)
    t = agg_ref[...] * inv
    dn0 = (((0,), (0,)), ((), ()))
    dn1 = (((1,), (1,)), ((), ()))
    mean = lax.dot_general(t, _eye(), dn0, preferred_element_type=jnp.float32)
    h = mean + lax.dot_general(x_ref[...], wr_ref[...], dn1,
                               preferred_element_type=jnp.float32)
    h = jnp.maximum(h + b_ref[...], 0.0)
    h_ref[...] = h
    y2 = lax.dot_general(w2l_ref[...], h, dn1,
                         preferred_element_type=jnp.float32)
    y2_ref[...] = _pack_block(y2)


def _mid(aggT, cpart, x, W_r, b, W2_l):
    return pl.pallas_call(
        _mid_body,
        grid=(NP // BLK,),
        in_specs=[
            pl.BlockSpec((D, BLK), lambda i: (0, i)),
            pl.BlockSpec((NW, BLK), lambda i: (0, i)),
            pl.BlockSpec((BLK, D), lambda i: (i, 0)),
            pl.BlockSpec((D, D), lambda i: (0, 0)),
            pl.BlockSpec((1, D), lambda i: (0, 0)),
            pl.BlockSpec((D, D), lambda i: (0, 0)),
        ],
        out_specs=[
            pl.BlockSpec((BLK, D), lambda i: (i, 0)),
            pl.BlockSpec((D // 2, BLK), lambda i: (0, i)),
        ],
        out_shape=[
            jax.ShapeDtypeStruct((N, D), jnp.float32),
            jax.ShapeDtypeStruct((D // 2, NP), jnp.int32),
        ],
    )(aggT, cpart, x, W_r, b.reshape(1, D), W2_l)


def _post_body(agg_ref, cp_ref, x_ref, wr_ref, b_ref, o_ref):
    cnt = jnp.sum(cp_ref[...], axis=0, keepdims=True)
    inv = 1.0 / jnp.maximum(cnt, 1.0)
    t = agg_ref[...] * inv
    dn0 = (((0,), (0,)), ((), ()))
    dn1 = (((1,), (1,)), ((), ()))
    mean = lax.dot_general(t, _eye(), dn0, preferred_element_type=jnp.float32)
    h = mean + lax.dot_general(x_ref[...], wr_ref[...], dn1,
                               preferred_element_type=jnp.float32)
    o_ref[...] = jnp.maximum(h + b_ref[...], 0.0)


def _post(aggT, cpart, h1, W_r, b):
    return pl.pallas_call(
        _post_body,
        grid=(NP // BLK,),
        in_specs=[
            pl.BlockSpec((D, BLK), lambda i: (0, i)),
            pl.BlockSpec((NW, BLK), lambda i: (0, i)),
            pl.BlockSpec((BLK, D), lambda i: (i, 0)),
            pl.BlockSpec((D, D), lambda i: (0, 0)),
            pl.BlockSpec((1, D), lambda i: (0, 0)),
        ],
        out_specs=pl.BlockSpec((BLK, D), lambda i: (i, 0)),
        out_shape=jax.ShapeDtypeStruct((N, D), jnp.float32),
    )(aggT, cpart, h1, W_r, b.reshape(1, D))


def kernel(x, edge_index, W1_l, b1_l, W1_r, W2_l, b2_l, W2_r):
    y1p, edges2d = _pre(W1_l, x, edge_index.astype(jnp.int32))
    edges = edges2d.reshape(E)
    agg1T, cpart = _sc_agg_counts(y1p, edges)
    h1, y2p = _mid(agg1T, cpart, x, W1_r, b1_l, W2_l)
    agg2T = _sc_agg(y2p, edges)
    return _post(agg2T, cpart, h1, W2_r, b2_l)
